# Initial kernel scaffold; baseline (speedup 1.0000x reference)
import jax
import jax.numpy as jnp
from jax.experimental import pallas as pl


def kernel(sentence_repr, entity_span_indices, W, b):
    raise NotImplementedError("write your pallas kernel here")


# speedup vs baseline: 14.9503x; 14.9503x over previous
import functools

import jax
import jax.numpy as jnp
from jax import lax
from jax.experimental import pallas as pl
from jax.experimental.pallas import tpu as pltpu
from jax.experimental.pallas import tpu_sc as plsc

B, S, D, N = 8, 512, 768, 32
NSPANS = B * N
NW = 32
SPW = NSPANS // NW
CH = 16
DV = 16
NVD = D // DV
DB = 128
NDB = D // DB
VPB = DB // DV
NEG = -3.0e38


def _sc_pool(x1d, starts, ends, rcp):
    mesh = plsc.VectorSubcoreMesh(core_axis_name="c", subcore_axis_name="s")

    @functools.partial(
        pl.kernel,
        mesh=mesh,
        out_type=jax.ShapeDtypeStruct((NSPANS * 2 * D,), jnp.float32),
        scratch_types=[
            pltpu.VMEM((CH * D,), jnp.float32),
            pltpu.VMEM((2 * D,), jnp.float32),
            pltpu.VMEM((24,), jnp.int32),
            pltpu.VMEM((24,), jnp.int32),
            pltpu.VMEM((528,), jnp.float32),
        ],
    )
    def kern(x_hbm, st_hbm, en_hbm, rcp_hbm, out_hbm, buf, acc, stv, env, rcpv):
        wid = lax.axis_index("s") * 2 + lax.axis_index("c")
        base = wid * SPW
        pltpu.sync_copy(st_hbm.at[pl.ds(base, SPW)], stv.at[pl.ds(0, SPW)])
        pltpu.sync_copy(en_hbm.at[pl.ds(base, SPW)], env.at[pl.ds(0, SPW)])
        pltpu.sync_copy(rcp_hbm, rcpv)
        def span_body(k, _):
            start = stv[pl.ds(k, 16)][0]
            end = env[pl.ds(k, 16)][0]
            s_id = base + k
            brow = (s_id // N) * S
            n_rows = end - start + 1
            full = n_rows // CH
            rem = n_rows - full * CH

            def init_body(j, _):
                val = jnp.where(j < NVD, NEG, jnp.float32(0.0))
                acc[pl.ds(j * DV, DV)] = jnp.full((DV,), jnp.float32(0.0)) + val
                return 0

            lax.fori_loop(0, 2 * NVD, init_body, 0)

            def chunk_body(c, _):
                r0 = start + c * CH
                pltpu.sync_copy(x_hbm.at[pl.ds((brow + r0) * D, CH * D)], buf)

                def db_body(db, _):
                    col = db * DB
                    accm = [acc[pl.ds(col + i * DV, DV)] for i in range(VPB)]
                    accs = [acc[pl.ds(D + col + i * DV, DV)] for i in range(VPB)]
                    for r in range(CH):
                        for i in range(VPB):
                            xv = buf[pl.ds(r * D + col + i * DV, DV)]
                            accm[i] = jnp.maximum(accm[i], xv)
                            accs[i] = accs[i] + xv
                    for i in range(VPB):
                        acc[pl.ds(col + i * DV, DV)] = accm[i]
                        acc[pl.ds(D + col + i * DV, DV)] = accs[i]
                    return 0

                lax.fori_loop(0, NDB, db_body, 0)
                return 0

            lax.fori_loop(0, full, chunk_body, 0)

            @pl.when(rem > 0)
            def _tail():
                r0t = jnp.maximum(end - (CH - 1), 0)
                t0 = start + full * CH
                pltpu.sync_copy(x_hbm.at[pl.ds((brow + r0t) * D, CH * D)], buf)

                def db_tail(db, _):
                    col = db * DB
                    accm = [acc[pl.ds(col + i * DV, DV)] for i in range(VPB)]
                    accs = [acc[pl.ds(D + col + i * DV, DV)] for i in range(VPB)]
                    for r in range(CH):
                        absrow = r0t + r
                        vm = jnp.logical_and(absrow >= start, absrow <= end)
                        vs = jnp.logical_and(absrow >= t0, absrow <= end)
                        for i in range(VPB):
                            xv = buf[pl.ds(r * D + col + i * DV, DV)]
                            accm[i] = jnp.maximum(accm[i], jnp.where(vm, xv, NEG))
                            accs[i] = accs[i] + jnp.where(vs, xv, jnp.float32(0.0))
                    for i in range(VPB):
                        acc[pl.ds(col + i * DV, DV)] = accm[i]
                        acc[pl.ds(D + col + i * DV, DV)] = accs[i]
                    return 0

                lax.fori_loop(0, NDB, db_tail, 0)

            scale = rcpv[pl.ds(n_rows - 1, 16)][0]

            def fin_body(j, _):
                acc[pl.ds(D + j * DV, DV)] = acc[pl.ds(D + j * DV, DV)] * scale
                return 0

            lax.fori_loop(0, NVD, fin_body, 0)
            pltpu.sync_copy(acc, out_hbm.at[pl.ds(s_id * 2 * D, 2 * D)])
            return 0

        lax.fori_loop(0, SPW, span_body, 0)

    return kern(x1d, starts, ends, rcp)


def _tc_proj(cat, W, b2):
    def body(c_ref, w_ref, b_ref, o_ref):
        o_ref[...] = lax.dot_general(
            c_ref[...], w_ref[...],
            dimension_numbers=(((1,), (1,)), ((), ())),
            preferred_element_type=jnp.float32,
        ) + b_ref[...]

    return pl.pallas_call(
        body,
        out_shape=jax.ShapeDtypeStruct((NSPANS, D), jnp.float32),
    )(cat, W, b2)


def kernel(sentence_repr, entity_span_indices, W, b):
    x1d = sentence_repr.reshape(B * S * D)
    esi = entity_span_indices.astype(jnp.int32).reshape(NSPANS, 2)
    rcp = 1.0 / jnp.arange(1, 529, dtype=jnp.float32)
    cat = _sc_pool(x1d, esi[:, 0], esi[:, 1], rcp).reshape(NSPANS, 2 * D)
    out = _tc_proj(cat, W, b.reshape(1, D))
    return out.reshape(B, N, D)


# speedup vs baseline: 27.2776x; 1.8246x over previous
import functools

import jax
import jax.numpy as jnp
from jax import lax
from jax.experimental import pallas as pl
from jax.experimental.pallas import tpu as pltpu
from jax.experimental.pallas import tpu_sc as plsc

B, S, D, N = 8, 512, 768, 32
NSPANS = B * N
NW = 32
SPW = NSPANS // NW
CH = 16
DV = 16
NVD = D // DV
DB = 128
NDB = D // DB
VPB = DB // DV
NEG = -3.0e38


def _sc_pool(x1d, starts, ends, rcp):
    mesh = plsc.VectorSubcoreMesh(core_axis_name="c", subcore_axis_name="s")

    @functools.partial(
        pl.kernel,
        mesh=mesh,
        out_type=jax.ShapeDtypeStruct((NSPANS * 2 * D,), jnp.float32),
        scratch_types=[
            pltpu.VMEM((2 * CH * D,), jnp.float32),
            pltpu.VMEM((2 * D,), jnp.float32),
            pltpu.VMEM((24,), jnp.int32),
            pltpu.VMEM((24,), jnp.int32),
            pltpu.VMEM((528,), jnp.float32),
            pltpu.SemaphoreType.DMA,
            pltpu.SemaphoreType.DMA,
        ],
    )
    def kern(x_hbm, st_hbm, en_hbm, rcp_hbm, out_hbm, buf, acc, stv, env, rcpv,
             sem0, sem1):
        wid = lax.axis_index("s") * 2 + lax.axis_index("c")
        base = wid * SPW
        pltpu.sync_copy(st_hbm.at[pl.ds(base, SPW)], stv.at[pl.ds(0, SPW)])
        pltpu.sync_copy(en_hbm.at[pl.ds(base, SPW)], env.at[pl.ds(0, SPW)])
        pltpu.sync_copy(rcp_hbm, rcpv)

        def span_body(k, _):
            start = stv[pl.ds(k, 16)][0]
            end = env[pl.ds(k, 16)][0]
            s_id = base + k
            brow = (s_id // N) * S
            n_rows = end - start + 1
            full = n_rows // CH
            rem = n_rows - full * CH
            total = full + jnp.where(rem > 0, 1, 0)
            r0t = jnp.maximum(end - (CH - 1), 0)
            t0 = start + full * CH

            def r0_of(c):
                return jnp.where(c < full, start + c * CH, r0t)

            def issue(c):
                src = x_hbm.at[pl.ds((brow + r0_of(c)) * D, CH * D)]

                @pl.when(c % 2 == 0)
                def _():
                    pltpu.async_copy(src, buf.at[pl.ds(0, CH * D)], sem0)

                @pl.when(c % 2 == 1)
                def _():
                    pltpu.async_copy(src, buf.at[pl.ds(CH * D, CH * D)], sem1)

            def wait(c):
                dummy = x_hbm.at[pl.ds(0, CH * D)]

                @pl.when(c % 2 == 0)
                def _():
                    pltpu.make_async_copy(
                        dummy, buf.at[pl.ds(0, CH * D)], sem0).wait()

                @pl.when(c % 2 == 1)
                def _():
                    pltpu.make_async_copy(
                        dummy, buf.at[pl.ds(CH * D, CH * D)], sem1).wait()

            issue(0)

            def init_body(j, _):
                val = jnp.where(j < NVD, NEG, jnp.float32(0.0))
                acc[pl.ds(j * DV, DV)] = jnp.full((DV,), jnp.float32(0.0)) + val
                return 0

            lax.fori_loop(0, 2 * NVD, init_body, 0)

            def chunk_body(c, _):
                @pl.when(c + 1 < total)
                def _():
                    issue(c + 1)

                wait(c)
                boff = (c % 2) * (CH * D)

                @pl.when(c < full)
                def _hot():
                    def db_body(db, _):
                        col = db * DB
                        accm = [acc[pl.ds(col + i * DV, DV)] for i in range(VPB)]
                        accs = [acc[pl.ds(D + col + i * DV, DV)] for i in range(VPB)]
                        for r in range(CH):
                            for i in range(VPB):
                                xv = buf[pl.ds(boff + r * D + col + i * DV, DV)]
                                accm[i] = jnp.maximum(accm[i], xv)
                                accs[i] = accs[i] + xv
                        for i in range(VPB):
                            acc[pl.ds(col + i * DV, DV)] = accm[i]
                            acc[pl.ds(D + col + i * DV, DV)] = accs[i]
                        return 0

                    lax.fori_loop(0, NDB, db_body, 0)

                @pl.when(c >= full)
                def _tail():
                    def db_tail(db, _):
                        col = db * DB
                        accm = [acc[pl.ds(col + i * DV, DV)] for i in range(VPB)]
                        accs = [acc[pl.ds(D + col + i * DV, DV)] for i in range(VPB)]
                        for r in range(CH):
                            absrow = r0t + r
                            vm = jnp.logical_and(absrow >= start, absrow <= end)
                            vs = jnp.logical_and(absrow >= t0, absrow <= end)
                            for i in range(VPB):
                                xv = buf[pl.ds(boff + r * D + col + i * DV, DV)]
                                accm[i] = jnp.maximum(accm[i], jnp.where(vm, xv, NEG))
                                accs[i] = accs[i] + jnp.where(vs, xv, jnp.float32(0.0))
                        for i in range(VPB):
                            acc[pl.ds(col + i * DV, DV)] = accm[i]
                            acc[pl.ds(D + col + i * DV, DV)] = accs[i]
                        return 0

                    lax.fori_loop(0, NDB, db_tail, 0)

                return 0

            lax.fori_loop(0, total, chunk_body, 0)

            scale = rcpv[pl.ds(n_rows - 1, 16)][0]

            def fin_body(j, _):
                acc[pl.ds(D + j * DV, DV)] = acc[pl.ds(D + j * DV, DV)] * scale
                return 0

            lax.fori_loop(0, NVD, fin_body, 0)
            pltpu.sync_copy(acc, out_hbm.at[pl.ds(s_id * 2 * D, 2 * D)])
            return 0

        lax.fori_loop(0, SPW, span_body, 0)

    return kern(x1d, starts, ends, rcp)


def _tc_proj(cat, W, b2):
    def body(c_ref, w_ref, b_ref, o_ref):
        o_ref[...] = lax.dot_general(
            c_ref[...], w_ref[...],
            dimension_numbers=(((1,), (1,)), ((), ())),
            preferred_element_type=jnp.float32,
        ) + b_ref[...]

    return pl.pallas_call(
        body,
        out_shape=jax.ShapeDtypeStruct((NSPANS, D), jnp.float32),
    )(cat, W, b2)


def kernel(sentence_repr, entity_span_indices, W, b):
    x1d = sentence_repr.reshape(B * S * D)
    esi = entity_span_indices.astype(jnp.int32).reshape(NSPANS, 2)
    rcp = 1.0 / jnp.arange(1, 529, dtype=jnp.float32)
    cat = _sc_pool(x1d, esi[:, 0], esi[:, 1], rcp).reshape(NSPANS, 2 * D)
    out = _tc_proj(cat, W, b.reshape(1, D))
    return out.reshape(B, N, D)
